# baseline (speedup 1.0000x reference)
import functools

import jax
import jax.numpy as jnp
from jax import lax
from jax.experimental import pallas as pl
from jax.experimental.pallas import tpu as pltpu
from jax.experimental.pallas import tpu_sc as plsc

_NUM_ROWS = 1000000
_D = 64
_B = 16384

_INFO = plsc.get_sparse_core_info()
_NC = _INFO.num_cores
_NS = _INFO.num_subcores
_NW = _NC * _NS
_BPW = _B // _NW
_CHUNK = 128
_NCHUNK = _BPW // _CHUNK
_L = 16
_NV = _D // _L
_EPS = 1e-5


def _ln_body(ids_hbm, table_hbm, gamma_hbm, beta_hbm, out_hbm,
             idx_v, rows_v, gam_v, bet_v, sem):
    wid = lax.axis_index("s") * _NC + lax.axis_index("c")
    base = wid * _BPW

    for j in range(_NCHUNK):
        pltpu.sync_copy(ids_hbm.at[pl.ds(base + j * _CHUNK, _CHUNK)],
                        idx_v.at[j])
    pltpu.sync_copy(gamma_hbm, gam_v)
    pltpu.sync_copy(beta_hbm, bet_v)

    copies = [
        pltpu.async_copy(table_hbm.at[idx_v.at[j]],
                         rows_v.at[pl.ds(j * _CHUNK, _CHUNK)], sem)
        for j in range(_NCHUNK)
    ]
    for cp in copies:
        cp.wait()

    g = [gam_v[pl.ds(k * _L, _L)] for k in range(_NV)]
    b = [bet_v[pl.ds(k * _L, _L)] for k in range(_NV)]
    inv_d = jnp.float32(1.0 / _D)

    def row(r, carry):
        x = [rows_v[r, pl.ds(k * _L, _L)] for k in range(_NV)]
        mean = jnp.sum(x[0] + x[1] + x[2] + x[3]) * inv_d
        d = [xk - mean for xk in x]
        var = jnp.sum(d[0] * d[0] + d[1] * d[1] + d[2] * d[2] + d[3] * d[3]) * inv_d
        t = jnp.broadcast_to(var + jnp.float32(_EPS), (_L,))
        y = plsc.bitcast(
            jnp.int32(0x5F3759DF) - (plsc.bitcast(t, jnp.int32) >> 1),
            jnp.float32)
        half_t = t * jnp.float32(0.5)
        for _ in range(3):
            y = y * (jnp.float32(1.5) - half_t * y * y)
        for k in range(_NV):
            rows_v[r, pl.ds(k * _L, _L)] = d[k] * y * g[k] + b[k]
        return carry

    lax.fori_loop(0, _BPW, row, 0, unroll=4)

    pltpu.sync_copy(rows_v, out_hbm.at[pl.ds(base, _BPW)])


@jax.jit
def _ln_embed(ids, table, gamma, beta):
    mesh = plsc.VectorSubcoreMesh(core_axis_name="c", subcore_axis_name="s")
    return pl.kernel(
        _ln_body,
        out_type=jax.ShapeDtypeStruct((_B, _D), jnp.float32),
        mesh=mesh,
        compiler_params=pltpu.CompilerParams(
            needs_layout_passes=False, use_tc_tiling_on_sc=False),
        scratch_types=[
            pltpu.VMEM((_NCHUNK, _CHUNK), jnp.int32),
            pltpu.VMEM((_BPW, _D), jnp.float32),
            pltpu.VMEM((_D,), jnp.float32),
            pltpu.VMEM((_D,), jnp.float32),
            pltpu.SemaphoreType.DMA,
        ],
    )(ids, table, gamma, beta)


def kernel(composer_ids, table, ln_gamma, ln_beta):
    ids = composer_ids.astype(jnp.int32)
    return _ln_embed(ids, table, ln_gamma, ln_beta)
